# Initial kernel scaffold; baseline (speedup 1.0000x reference)
import jax
import jax.numpy as jnp
from jax.experimental import pallas as pl


def kernel(risk_pred, time, event):
    raise NotImplementedError("write your pallas kernel here")


# speedup vs baseline: 3.0868x; 3.0868x over previous
import functools

import jax
import jax.numpy as jnp
from jax import lax
from jax.experimental import pallas as pl
from jax.experimental.pallas import tpu as pltpu
from jax.experimental.pallas import tpu_sc as plsc

N = 16384
M = 32
L = 16
NV = N // L
RADIX_BITS = 10
RADIX = 1 << RADIX_BITS
HIST_V = RADIX // L

_LN2 = 0.6931471805599453
_SQRT2 = 1.4142135623730951


def _log_poly(x):
  bits = plsc.bitcast(x, jnp.int32)
  e = jnp.right_shift(bits, 23) - 127
  m = plsc.bitcast(
      jnp.bitwise_or(jnp.bitwise_and(bits, 0x7FFFFF), 0x3F800000),
      jnp.float32)
  big = m > _SQRT2
  m = jnp.where(big, m * 0.5, m)
  e = e + jnp.where(big, 1, 0)
  s = (m - 1.0) / (m + 1.0)
  s2 = s * s
  p = 1.0 + s2 * (1.0 / 3.0 + s2 * (0.2 + s2 * (1.0 / 7.0 + s2 / 9.0)))
  return e.astype(jnp.float32) * _LN2 + 2.0 * s * p


def _sc_body(time_hbm, risk_hbm, ev_hbm, out_hbm,
             time_c, risk_c, ev_c, key_a, key_b, idx_a, idx_b,
             hist, off, pvec):
  wid = lax.axis_index("s") * 2 + lax.axis_index("c")

  pltpu.sync_copy(time_hbm.at[wid], time_c)
  pltpu.sync_copy(risk_hbm.at[wid], risk_c)
  pltpu.sync_copy(ev_hbm.at[wid], ev_c)

  def keygen(i, carry):
    maxv, s1v, sev = carry
    sl = pl.ds(i * L, L)
    t = time_c[sl]
    r = risk_c[sl]
    e = ev_c[sl]
    key_a[sl] = 0x3F7FFFFF - plsc.bitcast(t, jnp.int32)
    return (jnp.maximum(maxv, r), s1v + r * e, sev + e)

  zero = jnp.zeros((L,), jnp.float32)
  maxv, s1v, sev = lax.fori_loop(
      0, NV, keygen, (jnp.full((L,), -jnp.inf, jnp.float32), zero, zero))
  gamma = jnp.max(maxv)

  lane_iota = lax.iota(jnp.int32, L)

  def radix_pass(shift, src_key, dst_key, src_idx, dst_idx):
    def clear(j, _):
      hist[pl.ds(j * L, L)] = jnp.zeros((L,), jnp.int32)
      return 0
    lax.fori_loop(0, HIST_V, clear, 0)

    def hist_body(i, _):
      k = src_key[pl.ds(i * L, L)]
      d = jnp.bitwise_and(jnp.right_shift(k, shift), RADIX - 1)
      occ, last = plsc.scan_count(d)
      plsc.addupdate_scatter(hist, [d], occ, mask=last)
      return 0
    lax.fori_loop(0, NV, hist_body, 0)

    def scan_body(j, carry):
      sl = pl.ds(j * L, L)
      h = hist[sl]
      inc = plsc.cumsum(h)
      off[sl] = inc - h + carry
      return carry + jnp.sum(h)
    lax.fori_loop(0, HIST_V, scan_body, jnp.int32(0))

    def perm_body(i, _):
      k = src_key[pl.ds(i * L, L)]
      d = jnp.bitwise_and(jnp.right_shift(k, shift), RADIX - 1)
      occ, last = plsc.scan_count(d)
      base = plsc.load_gather(off, [d])
      pos = base + occ - 1
      if dst_key is not None:
        plsc.store_scatter(dst_key, [pos], k)
      if src_idx is None:
        iv = i * L + lane_iota
      else:
        iv = src_idx[pl.ds(i * L, L)]
      plsc.store_scatter(dst_idx, [pos], iv)
      plsc.store_scatter(off, [d], base + occ, mask=last)
      return 0
    lax.fori_loop(0, NV, perm_body, 0)

  radix_pass(0, key_a, key_b, None, idx_b)
  radix_pass(RADIX_BITS, key_b, key_a, idx_b, idx_a)
  radix_pass(2 * RADIX_BITS, key_a, None, idx_a, idx_b)

  def cox_body(i, carry):
    c0, acc2 = carry
    iv = idx_b[pl.ds(i * L, L)]
    r = plsc.load_gather(risk_c, [iv])
    e = plsc.load_gather(ev_c, [iv])
    x = jnp.exp(r - gamma)
    cs = plsc.cumsum(x) + c0
    lg = _log_poly(cs + 1e-10)
    return (c0 + jnp.sum(x), acc2 + e * lg)

  c0, acc2 = lax.fori_loop(0, NV, cox_body, (jnp.float32(0.0), zero))

  pvec[...] = s1v - acc2 - gamma * sev
  pltpu.sync_copy(pvec, out_hbm.at[wid])


@jax.jit
def _cox_loss(time_t, risk_t, ev_t):
  mesh = plsc.VectorSubcoreMesh(core_axis_name="c", subcore_axis_name="s")
  f = pl.kernel(
      _sc_body,
      out_type=jax.ShapeDtypeStruct((M, L), jnp.float32),
      mesh=mesh,
      scratch_types=[
          pltpu.VMEM((N,), jnp.float32),
          pltpu.VMEM((N,), jnp.float32),
          pltpu.VMEM((N,), jnp.float32),
          pltpu.VMEM((N,), jnp.int32),
          pltpu.VMEM((N,), jnp.int32),
          pltpu.VMEM((N,), jnp.int32),
          pltpu.VMEM((N,), jnp.int32),
          pltpu.VMEM((RADIX,), jnp.int32),
          pltpu.VMEM((RADIX,), jnp.int32),
          pltpu.VMEM((L,), jnp.float32),
      ],
      compiler_params=pltpu.CompilerParams(needs_layout_passes=False),
  )
  out = f(time_t, risk_t, ev_t)
  return -(jnp.sum(out) / (N * M))


def kernel(risk_pred, time, event):
  return _cox_loss(time.T, risk_pred.T, event.T)


# speedup vs baseline: 4.0771x; 1.3208x over previous
import jax
import jax.numpy as jnp
from jax import lax
from jax.experimental import pallas as pl
from jax.experimental.pallas import tpu as pltpu
from jax.experimental.pallas import tpu_sc as plsc

N = 16384
M = 32
L = 16
NV = N // L
RADIX_BITS = 10
RADIX = 1 << RADIX_BITS
HIST_V = RADIX // L

_LN2 = 0.6931471805599453
_SQRT2 = 1.4142135623730951
_LANE15 = 15


def _log_poly(x):
  bits = plsc.bitcast(x, jnp.int32)
  e = jnp.right_shift(bits, 23) - 127
  m = plsc.bitcast(
      jnp.bitwise_or(jnp.bitwise_and(bits, 0x7FFFFF), 0x3F800000),
      jnp.float32)
  big = m > _SQRT2
  m = jnp.where(big, m * 0.5, m)
  e = e + jnp.where(big, 1, 0)
  s = (m - 1.0) / (m + 1.0)
  s2 = s * s
  p = 1.0 + s2 * (1.0 / 3.0 + s2 * (0.2 + s2 * (1.0 / 7.0 + s2 / 9.0)))
  return e.astype(jnp.float32) * _LN2 + 2.0 * s * p


def _digit(k, shift):
  return jnp.bitwise_and(jnp.right_shift(k, shift), RADIX - 1)


def _sc_body(time_hbm, risk_hbm, ev_hbm, out_hbm,
             time_c, risk_c, ev_c, key_a, key_b, idx_a, idx_b,
             hist_a, hist_b, off, pvec):
  wid = lax.axis_index("s") * 2 + lax.axis_index("c")

  pltpu.sync_copy(time_hbm.at[wid], time_c)
  pltpu.sync_copy(risk_hbm.at[wid], risk_c)
  pltpu.sync_copy(ev_hbm.at[wid], ev_c)

  zero_i = jnp.zeros((L,), jnp.int32)
  zero_f = jnp.zeros((L,), jnp.float32)
  lane_iota = lax.iota(jnp.int32, L)

  def clear(h_ref):
    def body(j, _):
      h_ref[pl.ds(j * L, L)] = zero_i
      return 0
    lax.fori_loop(0, HIST_V, body, 0, unroll=8)

  clear(hist_a)

  def keygen(i, carry):
    maxv, s1v, sev = carry
    sl = pl.ds(i * L, L)
    t = time_c[sl]
    r = risk_c[sl]
    e = ev_c[sl]
    k = 0x3F7FFFFF - plsc.bitcast(t, jnp.int32)
    key_a[sl] = k
    d = _digit(k, 0)
    occ, last = plsc.scan_count(d)
    plsc.addupdate_scatter(hist_a, [d], occ, mask=last)
    return (jnp.maximum(maxv, r), s1v + r * e, sev + e)

  maxv, s1v, sev = lax.fori_loop(
      0, NV, keygen, (jnp.full((L,), -jnp.inf, jnp.float32), zero_f, zero_f),
      unroll=4)
  gamma = jnp.max(maxv)

  def hist_scan(h_ref):
    def body(j, carry):
      sl = pl.ds(j * L, L)
      h = h_ref[sl]
      inc = plsc.cumsum(h)
      off[sl] = inc - h + carry
      return carry + jnp.sum(h)
    lax.fori_loop(0, HIST_V, body, jnp.int32(0), unroll=4)

  def radix_pass(shift, src_key, dst_key, src_idx, dst_idx, h_ref, h_next):
    hist_scan(h_ref)
    if h_next is not None:
      clear(h_next)

    def perm_body(i, _):
      k = src_key[pl.ds(i * L, L)]
      d = _digit(k, shift)
      occ, last = plsc.scan_count(d)
      base = plsc.load_gather(off, [d])
      pos = base + occ - 1
      if dst_key is not None:
        plsc.store_scatter(dst_key, [pos], k)
      if src_idx is None:
        iv = i * L + lane_iota
      else:
        iv = src_idx[pl.ds(i * L, L)]
      plsc.store_scatter(dst_idx, [pos], iv)
      plsc.store_scatter(off, [d], base + occ, mask=last)
      if h_next is not None:
        d2 = _digit(k, shift + RADIX_BITS)
        occ2, last2 = plsc.scan_count(d2)
        plsc.addupdate_scatter(h_next, [d2], occ2, mask=last2)
      return 0
    lax.fori_loop(0, NV, perm_body, 0, unroll=4)

  radix_pass(0, key_a, key_b, None, idx_b, hist_a, hist_b)
  radix_pass(RADIX_BITS, key_b, key_a, idx_b, idx_a, hist_b, hist_a)
  radix_pass(2 * RADIX_BITS, key_a, None, idx_a, idx_b, hist_a, None)

  def cox_body(i, carry):
    c0, acc2 = carry
    iv = idx_b[pl.ds(i * L, L)]
    r = plsc.load_gather(risk_c, [iv])
    e = plsc.load_gather(ev_c, [iv])
    x = jnp.exp(r - gamma)
    cs_raw = plsc.cumsum(x)
    lg = _log_poly(cs_raw + c0 + 1e-10)
    return (c0 + jnp.sum(x), acc2 + e * lg)

  c0, acc2 = lax.fori_loop(0, NV, cox_body, (jnp.float32(0.0), zero_f),
                           unroll=4)

  pvec[...] = s1v - acc2 - gamma * sev
  pltpu.sync_copy(pvec, out_hbm.at[wid])


@jax.jit
def _cox_loss(time_t, risk_t, ev_t):
  mesh = plsc.VectorSubcoreMesh(core_axis_name="c", subcore_axis_name="s")
  f = pl.kernel(
      _sc_body,
      out_type=jax.ShapeDtypeStruct((M, L), jnp.float32),
      mesh=mesh,
      scratch_types=[
          pltpu.VMEM((N,), jnp.float32),
          pltpu.VMEM((N,), jnp.float32),
          pltpu.VMEM((N,), jnp.float32),
          pltpu.VMEM((N,), jnp.int32),
          pltpu.VMEM((N,), jnp.int32),
          pltpu.VMEM((N,), jnp.int32),
          pltpu.VMEM((N,), jnp.int32),
          pltpu.VMEM((RADIX,), jnp.int32),
          pltpu.VMEM((RADIX,), jnp.int32),
          pltpu.VMEM((RADIX,), jnp.int32),
          pltpu.VMEM((L,), jnp.float32),
      ],
      compiler_params=pltpu.CompilerParams(needs_layout_passes=False),
  )
  out = f(time_t, risk_t, ev_t)
  return -(jnp.sum(out) / (N * M))


def kernel(risk_pred, time, event):
  return _cox_loss(time.T, risk_pred.T, event.T)


# speedup vs baseline: 4.1311x; 1.0132x over previous
import jax
import jax.numpy as jnp
from jax import lax
from jax.experimental import pallas as pl
from jax.experimental.pallas import tpu as pltpu
from jax.experimental.pallas import tpu_sc as plsc

N = 16384
M = 32
L = 16
NV = N // L
R1_BITS = 12
R1 = 1 << R1_BITS
R23_BITS = 9
R23 = 1 << R23_BITS
IDX_BITS = 14
IDX_MASK = (1 << IDX_BITS) - 1

_LN2 = 0.6931471805599453
_SQRT2 = 1.4142135623730951


def _log_poly(x):
  bits = plsc.bitcast(x, jnp.int32)
  e = jnp.right_shift(bits, 23) - 127
  m = plsc.bitcast(
      jnp.bitwise_or(jnp.bitwise_and(bits, 0x7FFFFF), 0x3F800000),
      jnp.float32)
  big = m > _SQRT2
  m = jnp.where(big, m * 0.5, m)
  e = e + jnp.where(big, 1, 0)
  s = (m - 1.0) / (m + 1.0)
  s2 = s * s
  p = 1.0 + s2 * (1.0 / 3.0 + s2 * (0.2 + s2 * (1.0 / 7.0 + s2 / 9.0)))
  return e.astype(jnp.float32) * _LN2 + 2.0 * s * p


def _sc_body(inp_hbm, out_hbm,
             time_c, risk_c, ev_c, key_a, work_a, work_b,
             hist_a, hist_b, off, pvec):
  wid = lax.axis_index("s") * 2 + lax.axis_index("c")

  pltpu.sync_copy(inp_hbm.at[0, wid], time_c)
  pltpu.sync_copy(inp_hbm.at[1, wid], risk_c)
  pltpu.sync_copy(inp_hbm.at[2, wid], ev_c)

  zero_i = jnp.zeros((L,), jnp.int32)
  zero_f = jnp.zeros((L,), jnp.float32)
  lane_iota = lax.iota(jnp.int32, L)

  def clear(h_ref, nv):
    def body(j, _):
      h_ref[pl.ds(j * L, L)] = zero_i
      return 0
    lax.fori_loop(0, nv, body, 0, unroll=8)

  clear(hist_a, R1 // L)

  def keygen(i, carry):
    maxv, s1v, sev = carry
    sl = pl.ds(i * L, L)
    t = time_c[sl]
    r = risk_c[sl]
    e = ev_c[sl]
    k = 0x3F7FFFFF - plsc.bitcast(t, jnp.int32)
    key_a[sl] = k
    d = jnp.bitwise_and(k, R1 - 1)
    occ, last = plsc.scan_count(d)
    plsc.addupdate_scatter(hist_a, [d], occ, mask=last)
    return (jnp.maximum(maxv, r), s1v + r * e, sev + e)

  maxv, s1v, sev = lax.fori_loop(
      0, NV, keygen, (jnp.full((L,), -jnp.inf, jnp.float32), zero_f, zero_f),
      unroll=4)
  gamma = jnp.max(maxv)

  def hist_scan(h_ref, nv):
    def body(j, carry):
      sl = pl.ds(j * L, L)
      h = h_ref[sl]
      inc = plsc.cumsum(h)
      off[sl] = inc - h + carry
      return carry + jnp.sum(h)
    lax.fori_loop(0, nv, body, jnp.int32(0), unroll=4)

  hist_scan(hist_a, R1 // L)
  clear(hist_b, R23 // L)

  def perm1(i, _):
    k = key_a[pl.ds(i * L, L)]
    d = jnp.bitwise_and(k, R1 - 1)
    occ, last = plsc.scan_count(d)
    base = plsc.load_gather(off, [d])
    pos = base + occ - 1
    pack = jnp.bitwise_or(
        jnp.left_shift(jnp.right_shift(k, R1_BITS), IDX_BITS),
        i * L + lane_iota)
    plsc.store_scatter(work_b, [pos], pack)
    plsc.store_scatter(off, [d], base + occ, mask=last)
    d2 = jnp.bitwise_and(jnp.right_shift(pack, IDX_BITS), R23 - 1)
    occ2, last2 = plsc.scan_count(d2)
    plsc.addupdate_scatter(hist_b, [d2], occ2, mask=last2)
    return 0
  lax.fori_loop(0, NV, perm1, 0, unroll=4)

  hist_scan(hist_b, R23 // L)
  clear(hist_a, R23 // L)

  def perm2(i, _):
    pack = work_b[pl.ds(i * L, L)]
    d = jnp.bitwise_and(jnp.right_shift(pack, IDX_BITS), R23 - 1)
    occ, last = plsc.scan_count(d)
    base = plsc.load_gather(off, [d])
    pos = base + occ - 1
    plsc.store_scatter(work_a, [pos], pack)
    plsc.store_scatter(off, [d], base + occ, mask=last)
    d3 = jnp.bitwise_and(jnp.right_shift(pack, IDX_BITS + R23_BITS), R23 - 1)
    occ2, last2 = plsc.scan_count(d3)
    plsc.addupdate_scatter(hist_a, [d3], occ2, mask=last2)
    return 0
  lax.fori_loop(0, NV, perm2, 0, unroll=4)

  hist_scan(hist_a, R23 // L)

  def perm3(i, _):
    pack = work_a[pl.ds(i * L, L)]
    d = jnp.bitwise_and(jnp.right_shift(pack, IDX_BITS + R23_BITS), R23 - 1)
    occ, last = plsc.scan_count(d)
    base = plsc.load_gather(off, [d])
    pos = base + occ - 1
    plsc.store_scatter(work_b, [pos], pack)
    plsc.store_scatter(off, [d], base + occ, mask=last)
    return 0
  lax.fori_loop(0, NV, perm3, 0, unroll=4)

  def cox_body(i, carry):
    c0, acc2 = carry
    iv = jnp.bitwise_and(work_b[pl.ds(i * L, L)], IDX_MASK)
    r = plsc.load_gather(risk_c, [iv])
    e = plsc.load_gather(ev_c, [iv])
    x = jnp.exp(r - gamma)
    cs_raw = plsc.cumsum(x)
    lg = _log_poly(cs_raw + c0 + 1e-10)
    return (c0 + jnp.sum(x), acc2 + e * lg)

  c0, acc2 = lax.fori_loop(0, NV, cox_body, (jnp.float32(0.0), zero_f),
                           unroll=4)

  pvec[...] = s1v - acc2 - gamma * sev
  pltpu.sync_copy(pvec, out_hbm.at[wid])


@jax.jit
def _cox_loss(stacked):
  mesh = plsc.VectorSubcoreMesh(core_axis_name="c", subcore_axis_name="s")
  f = pl.kernel(
      _sc_body,
      out_type=jax.ShapeDtypeStruct((M, L), jnp.float32),
      mesh=mesh,
      scratch_types=[
          pltpu.VMEM((N,), jnp.float32),
          pltpu.VMEM((N,), jnp.float32),
          pltpu.VMEM((N,), jnp.float32),
          pltpu.VMEM((N,), jnp.int32),
          pltpu.VMEM((N,), jnp.int32),
          pltpu.VMEM((N,), jnp.int32),
          pltpu.VMEM((R1,), jnp.int32),
          pltpu.VMEM((R23,), jnp.int32),
          pltpu.VMEM((R1,), jnp.int32),
          pltpu.VMEM((L,), jnp.float32),
      ],
      compiler_params=pltpu.CompilerParams(needs_layout_passes=False),
  )
  out = f(stacked)
  return -(jnp.sum(out) / (N * M))


def kernel(risk_pred, time, event):
  stacked = jnp.stack([time.T, risk_pred.T, event.T])
  return _cox_loss(stacked)
